# baseline (speedup 1.0000x reference)
import functools

import jax
import jax.numpy as jnp
from jax import lax
from jax.experimental import pallas as pl
from jax.experimental.pallas import tpu as pltpu
from jax.experimental.pallas import tpu_sc as plsc

N_LEVELS = 16
HASHMAP_SIZE = 2 ** 19
GRID_SIZE = 512.0
MASK = HASHMAP_SIZE - 1
P2 = -1640531535
P3 = 805459861

N_POINTS = 262144
D_OUT = 2 * N_LEVELS
NC = 2
NS = 16
NW = NC * NS
BPW = N_POINTS // NW
CHUNK = 1024
GCH = 128
L = 16
RPS = HASHMAP_SIZE // NS
ILC = 1024


def _sc_encode(xflat, tabs):
    mesh = plsc.VectorSubcoreMesh(core_axis_name="c", subcore_axis_name="s")

    @functools.partial(
        pl.kernel,
        mesh=mesh,
        out_type=(
            jax.ShapeDtypeStruct((N_POINTS, D_OUT), jnp.float32),
            jax.ShapeDtypeStruct((HASHMAP_SIZE, D_OUT), jnp.float32),
        ),
        compiler_params=pltpu.CompilerParams(
            use_tc_tiling_on_sc=False, needs_layout_passes=False),
        scratch_types=[
            pltpu.VMEM((3 * BPW,), jnp.float32),
            pltpu.VMEM((BPW,), jnp.int32),
            pltpu.VMEM((CHUNK, D_OUT), jnp.float32),
            pltpu.SemaphoreType.DMA,
            pltpu.SemaphoreType.DMA,
        ],
    )
    def k(xflat_hbm, tabs_hbm, out_hbm, tabT_hbm, xv, idx_v, rows_v, xsem, sem):
        cid = lax.axis_index("c")
        sid = lax.axis_index("s")
        wid = sid * jnp.int32(NC) + cid
        base = wid * jnp.int32(BPW)

        xcp = pltpu.async_copy(
            xflat_hbm.at[pl.ds(base * jnp.int32(3), 3 * BPW)], xv, xsem)

        rbase = sid * jnp.int32(RPS)

        def il_body(j, carry):
            r0 = rbase + j * jnp.int32(ILC)
            cps = []
            for t in range(N_LEVELS):
                cps.append(pltpu.async_copy(
                    tabs_hbm.at[jnp.int32(t), pl.ds(r0, ILC)],
                    rows_v.at[:, pl.ds(2 * t, 2)],
                    sem,
                ))
            for c in cps:
                c.wait()
            pltpu.sync_copy(rows_v, tabT_hbm.at[pl.ds(r0, ILC)])
            return carry

        lax.fori_loop(jnp.int32(0), jnp.int32(RPS // ILC), il_body,
                      jnp.int32(0))

        xcp.wait()
        lane3 = lax.iota(jnp.int32, L) * jnp.int32(3)

        def hash_body(i, carry):
            s3 = i * jnp.int32(3 * L)
            i0 = lane3 + s3
            fx = plsc.load_gather(xv, [i0])
            fy = plsc.load_gather(xv, [i0 + jnp.int32(1)])
            fz = plsc.load_gather(xv, [i0 + jnp.int32(2)])
            cx = (fx * GRID_SIZE).astype(jnp.int32)
            cy = (fy * GRID_SIZE).astype(jnp.int32)
            cz = (fz * GRID_SIZE).astype(jnp.int32)
            h = (cx ^ (cy * jnp.int32(P2)) ^ (cz * jnp.int32(P3))) & jnp.int32(MASK)
            idx_v[pl.ds(i * jnp.int32(L), L)] = h
            return carry

        lax.fori_loop(jnp.int32(0), jnp.int32(BPW // L), hash_body,
                      jnp.int32(0))

        plsc.subcore_barrier()

        def chunk_body(j, carry):
            off = j * jnp.int32(CHUNK)
            cps = []
            for t in range(CHUNK // GCH):
                cps.append(pltpu.async_copy(
                    tabT_hbm.at[idx_v.at[pl.ds(off + jnp.int32(t * GCH), GCH)]],
                    rows_v.at[pl.ds(t * GCH, GCH)],
                    sem,
                ))
            for c in cps:
                c.wait()
            pltpu.sync_copy(rows_v, out_hbm.at[pl.ds(base + off, CHUNK)])
            return carry

        lax.fori_loop(jnp.int32(0), jnp.int32(BPW // CHUNK), chunk_body,
                      jnp.int32(0))

    return k(xflat, tabs)


def kernel(x, tables):
    out, _ = _sc_encode(x.reshape(-1), tables)
    return out


# speedup vs baseline: 32.3966x; 32.3966x over previous
import functools

import jax
import jax.numpy as jnp
from jax import lax
from jax.experimental import pallas as pl
from jax.experimental.pallas import tpu as pltpu
from jax.experimental.pallas import tpu_sc as plsc

N_LEVELS = 16
HASHMAP_SIZE = 2 ** 19
GRID_SIZE = 512.0
MASK = HASHMAP_SIZE - 1
P2 = -1640531535
P3 = 805459861

N_POINTS = 262144
D_OUT = 2 * N_LEVELS
NC = 2
NS = 16
NW = NC * NS
BPW = N_POINTS // NW
CHUNK = 1024
GCH = 128
L = 16


def _sc_encode(xc0, xc1, xc2, tabT):
    mesh = plsc.VectorSubcoreMesh(core_axis_name="c", subcore_axis_name="s")

    @functools.partial(
        pl.kernel,
        mesh=mesh,
        out_type=jax.ShapeDtypeStruct((N_POINTS, D_OUT), jnp.float32),
        compiler_params=pltpu.CompilerParams(
            use_tc_tiling_on_sc=False, needs_layout_passes=False),
        scratch_types=[
            pltpu.VMEM((BPW,), jnp.float32),
            pltpu.VMEM((BPW,), jnp.float32),
            pltpu.VMEM((BPW,), jnp.float32),
            pltpu.VMEM((BPW,), jnp.int32),
            pltpu.VMEM((CHUNK, D_OUT), jnp.float32),
            pltpu.SemaphoreType.DMA,
            pltpu.SemaphoreType.DMA,
        ],
    )
    def k(x0_hbm, x1_hbm, x2_hbm, tabT_hbm, out_hbm,
          xv0, xv1, xv2, idx_v, rows_v, xsem, sem):
        cid = lax.axis_index("c")
        sid = lax.axis_index("s")
        wid = sid * jnp.int32(NC) + cid
        base = wid * jnp.int32(BPW)

        cp0 = pltpu.async_copy(x0_hbm.at[pl.ds(base, BPW)], xv0, xsem)
        cp1 = pltpu.async_copy(x1_hbm.at[pl.ds(base, BPW)], xv1, xsem)
        cp2 = pltpu.async_copy(x2_hbm.at[pl.ds(base, BPW)], xv2, xsem)
        cp0.wait()
        cp1.wait()
        cp2.wait()

        def hash_body(i, carry):
            sl = pl.ds(i * jnp.int32(L), L)
            cx = (xv0[sl] * GRID_SIZE).astype(jnp.int32)
            cy = (xv1[sl] * GRID_SIZE).astype(jnp.int32)
            cz = (xv2[sl] * GRID_SIZE).astype(jnp.int32)
            h = (cx ^ (cy * jnp.int32(P2)) ^ (cz * jnp.int32(P3))) & jnp.int32(MASK)
            idx_v[sl] = h
            return carry

        lax.fori_loop(jnp.int32(0), jnp.int32(BPW // L), hash_body,
                      jnp.int32(0))

        def chunk_body(j, carry):
            off = j * jnp.int32(CHUNK)
            cps = []
            for t in range(CHUNK // GCH):
                cps.append(pltpu.async_copy(
                    tabT_hbm.at[idx_v.at[pl.ds(off + jnp.int32(t * GCH), GCH)]],
                    rows_v.at[pl.ds(t * GCH, GCH)],
                    sem,
                ))
            for c in cps:
                c.wait()
            pltpu.sync_copy(rows_v, out_hbm.at[pl.ds(base + off, CHUNK)])
            return carry

        lax.fori_loop(jnp.int32(0), jnp.int32(BPW // CHUNK), chunk_body,
                      jnp.int32(0))

    return k(xc0, xc1, xc2, tabT)


def kernel(x, tables):
    tabT = jnp.transpose(tables, (1, 0, 2)).reshape(HASHMAP_SIZE, D_OUT)
    return _sc_encode(x[:, 0], x[:, 1], x[:, 2], tabT)


# speedup vs baseline: 32.4018x; 1.0002x over previous
import functools

import jax
import jax.numpy as jnp
from jax import lax
from jax.experimental import pallas as pl
from jax.experimental.pallas import tpu as pltpu
from jax.experimental.pallas import tpu_sc as plsc

N_LEVELS = 16
HASHMAP_SIZE = 2 ** 19
GRID_SIZE = 512.0
MASK = HASHMAP_SIZE - 1
P2 = -1640531535
P3 = 805459861

N_POINTS = 262144
D_OUT = 2 * N_LEVELS
NC = 2
NS = 16
NW = NC * NS
BPW = N_POINTS // NW
CHUNK = 1024
GCH = 128
L = 16


def _sc_encode(xc0, xc1, xc2, tabT):
    mesh = plsc.VectorSubcoreMesh(core_axis_name="c", subcore_axis_name="s")

    @functools.partial(
        pl.kernel,
        mesh=mesh,
        out_type=jax.ShapeDtypeStruct((N_POINTS, D_OUT), jnp.float32),
        compiler_params=pltpu.CompilerParams(
            use_tc_tiling_on_sc=False, needs_layout_passes=True),
        scratch_types=[
            pltpu.VMEM((BPW,), jnp.float32),
            pltpu.VMEM((BPW,), jnp.float32),
            pltpu.VMEM((BPW,), jnp.float32),
            pltpu.VMEM((BPW,), jnp.int32),
            pltpu.VMEM((CHUNK, D_OUT), jnp.float32),
            pltpu.SemaphoreType.DMA,
            pltpu.SemaphoreType.DMA,
        ],
    )
    def k(x0_hbm, x1_hbm, x2_hbm, tabT_hbm, out_hbm,
          xv0, xv1, xv2, idx_v, rows_v, xsem, sem):
        cid = lax.axis_index("c")
        sid = lax.axis_index("s")
        wid = sid * jnp.int32(NC) + cid
        base = wid * jnp.int32(BPW)

        cp0 = pltpu.async_copy(x0_hbm.at[pl.ds(base, BPW)], xv0, xsem)
        cp1 = pltpu.async_copy(x1_hbm.at[pl.ds(base, BPW)], xv1, xsem)
        cp2 = pltpu.async_copy(x2_hbm.at[pl.ds(base, BPW)], xv2, xsem)
        cp0.wait()
        cp1.wait()
        cp2.wait()

        def hash_body(i, carry):
            sl = pl.ds(i * jnp.int32(L), L)
            cx = (xv0[sl] * GRID_SIZE).astype(jnp.int32)
            cy = (xv1[sl] * GRID_SIZE).astype(jnp.int32)
            cz = (xv2[sl] * GRID_SIZE).astype(jnp.int32)
            h = (cx ^ (cy * jnp.int32(P2)) ^ (cz * jnp.int32(P3))) & jnp.int32(MASK)
            idx_v[sl] = h
            return carry

        lax.fori_loop(jnp.int32(0), jnp.int32(BPW // L), hash_body,
                      jnp.int32(0))

        def chunk_body(j, carry):
            off = j * jnp.int32(CHUNK)
            cps = []
            for t in range(CHUNK // GCH):
                cps.append(pltpu.async_copy(
                    tabT_hbm.at[idx_v.at[pl.ds(off + jnp.int32(t * GCH), GCH)]],
                    rows_v.at[pl.ds(t * GCH, GCH)],
                    sem,
                ))
            for c in cps:
                c.wait()
            pltpu.sync_copy(rows_v, out_hbm.at[pl.ds(base + off, CHUNK)])
            return carry

        lax.fori_loop(jnp.int32(0), jnp.int32(BPW // CHUNK), chunk_body,
                      jnp.int32(0))

    return k(xc0, xc1, xc2, tabT)


def kernel(x, tables):
    tabT = jnp.transpose(tables, (1, 0, 2)).reshape(HASHMAP_SIZE, D_OUT)
    return _sc_encode(x[:, 0], x[:, 1], x[:, 2], tabT)
